# baseline (speedup 1.0000x reference)
import functools

import jax
import jax.numpy as jnp
from jax import lax
from jax.experimental import pallas as pl
from jax.experimental.pallas import tpu as pltpu
from jax.experimental.pallas import tpu_sc as plsc

B = 16
S = 32768
C = 64
K = 8
L = 16
NW = 32
GROUPS = (B * C) // L
GPW = GROUPS // NW
CPB = C // L
RB = 8
NB1 = S // RB
NB2 = NB1 // RB
NB3 = NB2 // RB
TS = 4096

_MESH = plsc.VectorSubcoreMesh(core_axis_name="c", subcore_axis_name="s")


def _insert8(rs, v):
    out = []
    for j in range(K):
        out.append(jnp.maximum(rs[j], v))
        v = jnp.minimum(rs[j], v)
    return tuple(out)


def _insert8_idx(vs, ids, v, vi):
    nvs, nids = [], []
    for j in range(K):
        c = v > vs[j]
        nvs.append(jnp.where(c, v, vs[j]))
        nids.append(jnp.where(c, vi, ids[j]))
        lo_v = jnp.where(c, vs[j], v)
        lo_i = jnp.where(c, ids[j], vi)
        v, vi = lo_v, lo_i
    return tuple(nvs), tuple(nids)


def _tc_blockmax(x3):
    def body(x_ref, o_ref):
        v = x_ref[0]
        o_ref[0] = jnp.max(v.reshape(TS // RB, RB, C), axis=1)

    return pl.pallas_call(
        body,
        grid=(B, S // TS),
        in_specs=[pl.BlockSpec((1, TS, C), lambda b, t: (b, t, 0))],
        out_specs=pl.BlockSpec((1, TS // RB, C), lambda b, t: (b, t, 0)),
        out_shape=jax.ShapeDtypeStruct((B, NB1, C), jnp.float32),
        compiler_params=pltpu.CompilerParams(
            dimension_semantics=("parallel", "arbitrary")),
    )(x3)


def _sc_select_fold(x2, g1_all):
    @functools.partial(
        pl.kernel,
        mesh=_MESH,
        out_type=jax.ShapeDtypeStruct((GROUPS, K, L), jnp.float32),
        compiler_params=pltpu.CompilerParams(
            use_tc_tiling_on_sc=False, needs_layout_passes=False),
        scratch_types=[
            pltpu.VMEM((NB1, L), jnp.float32),
            pltpu.VMEM((NB2, L), jnp.float32),
            pltpu.VMEM((NB3, L), jnp.float32),
            pltpu.VMEM((K, L), jnp.int32),
            pltpu.VMEM((K * RB * L, L), jnp.float32),
            pltpu.VMEM((K, L), jnp.float32),
            pltpu.SemaphoreType.DMA,
            pltpu.SemaphoreType.DMA,
        ],
    )
    def k(x_hbm, g1_hbm, out_hbm, g1, g2, g3, top_i, cand, top_v,
          sem0, semg):
        wid = lax.axis_index("s") * 2 + lax.axis_index("c")
        lane = lax.broadcasted_iota(jnp.int32, (L,), 0)
        neg = jnp.full((L,), -jnp.inf, jnp.float32)
        zero = jnp.zeros((L,), jnp.int32)

        def blockmax(src_ref, dst_ref, nblk):
            @plsc.parallel_loop(0, nblk, unroll=4)
            def _(ib_):
                base = ib_ * RB
                vs = [src_ref[base + r] for r in range(RB)]
                m01 = jnp.maximum(vs[0], vs[1])
                m23 = jnp.maximum(vs[2], vs[3])
                m45 = jnp.maximum(vs[4], vs[5])
                m67 = jnp.maximum(vs[6], vs[7])
                dst_ref[ib_] = jnp.maximum(
                    jnp.maximum(m01, m23), jnp.maximum(m45, m67))

        for gi in range(GPW):
            g = wid * GPW + gi
            b = g // CPB
            c0 = (g % CPB) * L
            row0 = b * S

            pltpu.sync_copy(g1_hbm.at[b, :, pl.ds(c0, L)], g1)
            blockmax(g1, g2, NB2)
            blockmax(g2, g3, NB3)

            def sel3(i, carry):
                vs, ids = carry
                return _insert8_idx(vs, ids, g3[i],
                                    jnp.full((L,), i, jnp.int32))

            vs, ids = lax.fori_loop(0, NB3, sel3, ((neg,) * K, (zero,) * K))

            for lvl_ref in (g2, g1):
                pids = ids
                vs, ids = (neg,) * K, (zero,) * K
                for j in range(K):
                    base = pids[j] * RB

                    def child(r, carry):
                        cvs, cids = carry
                        row = base + r
                        v = plsc.load_gather(lvl_ref, [row, lane])
                        return _insert8_idx(cvs, cids, v, row)

                    vs, ids = lax.fori_loop(0, RB, child, (vs, ids))

            for j in range(K):
                top_i[j] = ids[j]

            copies = []
            for j in range(K):
                idrow = top_i[j]
                for l in range(L):
                    blk = idrow[l]
                    copies.append(pltpu.async_copy(
                        x_hbm.at[pl.ds(row0 + blk * RB, RB),
                                 pl.ds(c0, L)],
                        cand.at[pl.ds((j * L + l) * RB, RB)], semg))
            for cp in copies:
                cp.wait()

            def fold(t, rs):
                j = t >> 3
                r = t & (RB - 1)
                row = j * (L * RB) + lane * RB + r
                v = plsc.load_gather(cand, [row, lane])
                return _insert8(rs, v)

            rs = lax.fori_loop(0, K * RB, fold, (neg,) * K)

            for j in range(K):
                top_v[j] = rs[j]
            pltpu.sync_copy(top_v, out_hbm.at[g])

    return k(x2, g1_all)


def kernel(inputs):
    x2 = inputs.reshape(B * S, C)
    g1_all = _tc_blockmax(inputs)
    out = _sc_select_fold(x2, g1_all)
    out = out.reshape(B, CPB, K, L).transpose(0, 1, 3, 2)
    return out.reshape(B, C * K)


# speedup vs baseline: 1.1562x; 1.1562x over previous
import functools

import jax
import jax.numpy as jnp
from jax import lax
from jax.experimental import pallas as pl
from jax.experimental.pallas import tpu as pltpu
from jax.experimental.pallas import tpu_sc as plsc

B = 16
S = 32768
C = 64
K = 8
L = 16
NW = 32
GROUPS = (B * C) // L
GPW = GROUPS // NW
CPB = C // L
RB = 32
NB1 = S // RB
FAN = 16
NB2 = NB1 // FAN
TS = 8192
HWIN = K // 2

_MESH = plsc.VectorSubcoreMesh(core_axis_name="c", subcore_axis_name="s")


def _insert8(rs, v):
    out = []
    for j in range(K):
        out.append(jnp.maximum(rs[j], v))
        v = jnp.minimum(rs[j], v)
    return tuple(out)


def _insert8_idx(vs, ids, v, vi):
    nvs, nids = [], []
    for j in range(K):
        c = v > vs[j]
        nvs.append(jnp.where(c, v, vs[j]))
        nids.append(jnp.where(c, vi, ids[j]))
        lo_v = jnp.where(c, vs[j], v)
        lo_i = jnp.where(c, ids[j], vi)
        v, vi = lo_v, lo_i
    return tuple(nvs), tuple(nids)


def _tc_blockmax(x3):
    def body(x_ref, o_ref):
        v = x_ref[0]
        o_ref[0] = jnp.max(v.reshape(TS // RB, RB, C), axis=1)

    return pl.pallas_call(
        body,
        grid=(B, S // TS),
        in_specs=[pl.BlockSpec((1, TS, C), lambda b, t: (b, t, 0))],
        out_specs=pl.BlockSpec((1, TS // RB, C), lambda b, t: (b, t, 0)),
        out_shape=jax.ShapeDtypeStruct((B, NB1, C), jnp.float32),
        compiler_params=pltpu.CompilerParams(
            dimension_semantics=("parallel", "arbitrary")),
    )(x3)


def _sc_select_fold(x2, g1_all):
    @functools.partial(
        pl.kernel,
        mesh=_MESH,
        out_type=jax.ShapeDtypeStruct((GROUPS, K, L), jnp.float32),
        compiler_params=pltpu.CompilerParams(
            use_tc_tiling_on_sc=False, needs_layout_passes=False),
        scratch_types=[
            pltpu.VMEM((NB1, L), jnp.float32),
            pltpu.VMEM((NB2, L), jnp.float32),
            pltpu.VMEM((K, L), jnp.int32),
            pltpu.VMEM((HWIN * L * RB, L), jnp.float32),
            pltpu.VMEM((K, L), jnp.float32),
            pltpu.SemaphoreType.DMA,
        ],
    )
    def k(x_hbm, g1_hbm, out_hbm, g1, g2, top_i, cand, top_v, semg):
        wid = lax.axis_index("s") * 2 + lax.axis_index("c")
        lane = lax.broadcasted_iota(jnp.int32, (L,), 0)
        neg = jnp.full((L,), -jnp.inf, jnp.float32)
        zero = jnp.zeros((L,), jnp.int32)

        for gi in range(GPW):
            g = wid * GPW + gi
            b = g // CPB
            c0 = (g % CPB) * L
            row0 = b * S

            pltpu.sync_copy(g1_hbm.at[b, :, pl.ds(c0, L)], g1)

            @plsc.parallel_loop(0, NB2, unroll=2)
            def _(ib_):
                base = ib_ * FAN
                vs = [g1[base + r] for r in range(FAN)]
                m = []
                for r in range(0, FAN, 2):
                    m.append(jnp.maximum(vs[r], vs[r + 1]))
                while len(m) > 1:
                    m = [jnp.maximum(m[i], m[i + 1])
                         for i in range(0, len(m), 2)]
                g2[ib_] = m[0]

            def sel2(i, carry):
                vs, ids = carry
                return _insert8_idx(vs, ids, g2[i],
                                    jnp.full((L,), i, jnp.int32))

            vs, ids = lax.fori_loop(0, NB2, sel2, ((neg,) * K, (zero,) * K))

            pids = ids
            vs, ids = (neg,) * K, (zero,) * K
            for j in range(K):
                base = pids[j] * FAN

                def child(r, carry):
                    cvs, cids = carry
                    row = base + r
                    v = plsc.load_gather(g1, [row, lane])
                    return _insert8_idx(cvs, cids, v, row)

                vs, ids = lax.fori_loop(0, FAN, child, (vs, ids))

            for j in range(K):
                top_i[j] = ids[j]

            rs_a, rs_b = (neg,) * K, (neg,) * K
            for jh in range(2):
                copies = []
                for jj in range(HWIN):
                    idrow = top_i[jh * HWIN + jj]
                    for l in range(L):
                        blk = idrow[l]
                        copies.append(pltpu.async_copy(
                            x_hbm.at[pl.ds(row0 + blk * RB, RB),
                                     pl.ds(c0, L)],
                            cand.at[pl.ds((jj * L + l) * RB, RB)], semg))
                for cp in copies:
                    cp.wait()

                def fold(t, carry):
                    ra, rb_ = carry
                    jj = t >> 4
                    r2 = (t & 15) * 2
                    base_ = jj * (L * RB) + lane * RB
                    va = plsc.load_gather(cand, [base_ + r2, lane])
                    vb = plsc.load_gather(cand, [base_ + r2 + 1, lane])
                    return _insert8(ra, va), _insert8(rb_, vb)

                rs_a, rs_b = lax.fori_loop(
                    0, HWIN * (RB // 2), fold, (rs_a, rs_b))

            rs = rs_a
            for j in range(K):
                rs = _insert8(rs, rs_b[j])

            for j in range(K):
                top_v[j] = rs[j]
            pltpu.sync_copy(top_v, out_hbm.at[g])

    return k(x2, g1_all)


def kernel(inputs):
    x2 = inputs.reshape(B * S, C)
    g1_all = _tc_blockmax(inputs)
    out = _sc_select_fold(x2, g1_all)
    out = out.reshape(B, CPB, K, L).transpose(0, 1, 3, 2)
    return out.reshape(B, C * K)


# speedup vs baseline: 1.6607x; 1.4364x over previous
import functools

import jax
import jax.numpy as jnp
from jax import lax
from jax.experimental import pallas as pl
from jax.experimental.pallas import tpu as pltpu
from jax.experimental.pallas import tpu_sc as plsc

B = 16
S = 32768
C = 64
K = 8
L = 16
NW = 32
GROUPS = (B * C) // L
GPW = GROUPS // NW
CPB = C // L
RB = 16
NB1 = S // RB
FAN = 16
NB2 = NB1 // FAN
TS = 8192
RPR = 2

_MESH = plsc.VectorSubcoreMesh(core_axis_name="c", subcore_axis_name="s")


def _insert8(rs, v):
    out = []
    for j in range(K):
        out.append(jnp.maximum(rs[j], v))
        v = jnp.minimum(rs[j], v)
    return tuple(out)


def _insert8_idx(vs, ids, v, vi):
    nvs, nids = [], []
    for j in range(K):
        c = v > vs[j]
        nvs.append(jnp.where(c, v, vs[j]))
        nids.append(jnp.where(c, vi, ids[j]))
        lo_v = jnp.where(c, vs[j], v)
        lo_i = jnp.where(c, ids[j], vi)
        v, vi = lo_v, lo_i
    return tuple(nvs), tuple(nids)


def _tc_blockmax(x3):
    def body(x_ref, o_ref):
        v = x_ref[0]
        o_ref[0] = jnp.max(v.reshape(TS // RB, RB, C), axis=1)

    return pl.pallas_call(
        body,
        grid=(B, S // TS),
        in_specs=[pl.BlockSpec((1, TS, C), lambda b, t: (b, t, 0))],
        out_specs=pl.BlockSpec((1, TS // RB, C), lambda b, t: (b, t, 0)),
        out_shape=jax.ShapeDtypeStruct((B, NB1, C), jnp.float32),
        compiler_params=pltpu.CompilerParams(
            dimension_semantics=("parallel", "arbitrary")),
    )(x3)


def _sc_select(g1_all):
    @functools.partial(
        pl.kernel,
        mesh=_MESH,
        out_type=jax.ShapeDtypeStruct((GROUPS, K, L), jnp.int32),
        compiler_params=pltpu.CompilerParams(
            use_tc_tiling_on_sc=False, needs_layout_passes=False),
        scratch_types=[
            pltpu.VMEM((NB1, L), jnp.float32),
            pltpu.VMEM((NB2, L), jnp.float32),
            pltpu.VMEM((K, L), jnp.int32),
        ],
    )
    def k(g1_hbm, bidx_hbm, g1, g2, top_i):
        wid = lax.axis_index("s") * 2 + lax.axis_index("c")
        lane = lax.broadcasted_iota(jnp.int32, (L,), 0)
        neg = jnp.full((L,), -jnp.inf, jnp.float32)
        zero = jnp.zeros((L,), jnp.int32)

        for gi in range(GPW):
            g = wid * GPW + gi
            b = g // CPB
            c0 = (g % CPB) * L

            pltpu.sync_copy(g1_hbm.at[b, :, pl.ds(c0, L)], g1)

            @plsc.parallel_loop(0, NB2, unroll=2)
            def _(ib_):
                base = ib_ * FAN
                m = [jnp.maximum(g1[base + r], g1[base + r + 1])
                     for r in range(0, FAN, 2)]
                while len(m) > 1:
                    m = [jnp.maximum(m[i], m[i + 1])
                         for i in range(0, len(m), 2)]
                g2[ib_] = m[0]

            def sel2(i, carry):
                vs, ids = carry
                return _insert8_idx(vs, ids, g2[i],
                                    jnp.full((L,), i, jnp.int32))

            vs, ids = lax.fori_loop(0, NB2, sel2, ((neg,) * K, (zero,) * K))

            pids = ids
            vs, ids = (neg,) * K, (zero,) * K
            for j in range(K):
                base = pids[j] * FAN

                def child(r, carry):
                    cvs, cids = carry
                    row = base + r
                    v = plsc.load_gather(g1, [row, lane])
                    return _insert8_idx(cvs, cids, v, row)

                vs, ids = lax.fori_loop(0, FAN, child, (vs, ids))

            for j in range(K):
                top_i[j] = ids[j]
            pltpu.sync_copy(top_i, bidx_hbm.at[g])

    return k(g1_all)


def _sc_refetch_fold(x2, bidx):
    @functools.partial(
        pl.kernel,
        mesh=_MESH,
        out_type=jax.ShapeDtypeStruct((GROUPS, K, L), jnp.float32),
        compiler_params=pltpu.CompilerParams(needs_layout_passes=False),
        scratch_types=[
            pltpu.VMEM((K, L), jnp.int32),
            pltpu.VMEM((RPR * L * RB, C), jnp.float32),
            pltpu.VMEM((K, L), jnp.float32),
            pltpu.SemaphoreType.DMA,
            pltpu.SemaphoreType.DMA,
        ],
    )
    def k(x_hbm, bidx_hbm, out_hbm, bidx_v, cand, top_v, semi, semg):
        wid = lax.axis_index("s") * 2 + lax.axis_index("c")
        lane = lax.broadcasted_iota(jnp.int32, (L,), 0)
        neg = jnp.full((L,), -jnp.inf, jnp.float32)

        for gi in range(GPW):
            g = wid * GPW + gi
            b = g // CPB
            c0 = (g % CPB) * L
            row0 = b * S

            pltpu.async_copy(bidx_hbm.at[g], bidx_v, semi).wait()

            rs_a, rs_b = (neg,) * K, (neg,) * K
            for rr in range(K // RPR):
                copies = []
                for jj in range(RPR):
                    idrow = bidx_v[rr * RPR + jj]
                    for l in range(L):
                        blk = idrow[l]
                        copies.append(pltpu.async_copy(
                            x_hbm.at[pl.ds(row0 + blk * RB, RB)],
                            cand.at[pl.ds((jj * L + l) * RB, RB)], semg))
                for cp in copies:
                    cp.wait()

                def fold(t, carry):
                    ra, rb_ = carry
                    jj = t >> 3
                    r2 = (t & 7) * 2
                    base_ = jj * (L * RB) + lane * RB
                    va = plsc.load_gather(cand, [base_ + r2, c0 + lane])
                    vb = plsc.load_gather(cand, [base_ + r2 + 1,
                                                 c0 + lane])
                    return _insert8(ra, va), _insert8(rb_, vb)

                rs_a, rs_b = lax.fori_loop(
                    0, RPR * (RB // 2), fold, (rs_a, rs_b))

            rs = rs_a
            for j in range(K):
                rs = _insert8(rs, rs_b[j])

            for j in range(K):
                top_v[j] = rs[j]
            pltpu.sync_copy(top_v, out_hbm.at[g])

    return k(x2, bidx)


def kernel(inputs):
    x2 = inputs.reshape(B * S, C)
    g1_all = _tc_blockmax(inputs)
    bidx = _sc_select(g1_all)
    out = _sc_refetch_fold(x2, bidx)
    out = out.reshape(B, CPB, K, L).transpose(0, 1, 3, 2)
    return out.reshape(B, C * K)


# speedup vs baseline: 1.6638x; 1.0019x over previous
import functools

import jax
import jax.numpy as jnp
from jax import lax
from jax.experimental import pallas as pl
from jax.experimental.pallas import tpu as pltpu
from jax.experimental.pallas import tpu_sc as plsc

B = 16
S = 32768
C = 64
K = 8
L = 16
NW = 32
GROUPS = (B * C) // L
GPW = GROUPS // NW
CPB = C // L
RB = 16
NB1 = S // RB
FAN = 16
NB2 = NB1 // FAN
TS = 8192
RPR = 2

_MESH = plsc.VectorSubcoreMesh(core_axis_name="c", subcore_axis_name="s")


def _insert8(rs, v):
    out = []
    for j in range(K):
        out.append(jnp.maximum(rs[j], v))
        v = jnp.minimum(rs[j], v)
    return tuple(out)


def _insert8_idx(vs, ids, v, vi):
    nvs, nids = [], []
    for j in range(K):
        c = v > vs[j]
        nvs.append(jnp.where(c, v, vs[j]))
        nids.append(jnp.where(c, vi, ids[j]))
        lo_v = jnp.where(c, vs[j], v)
        lo_i = jnp.where(c, ids[j], vi)
        v, vi = lo_v, lo_i
    return tuple(nvs), tuple(nids)


def _tc_blockmax(x3):
    def body(x_ref, o_ref):
        v = x_ref[0]
        o_ref[0] = jnp.max(v.reshape(TS // RB, RB, C), axis=1)

    return pl.pallas_call(
        body,
        grid=(B, S // TS),
        in_specs=[pl.BlockSpec((1, TS, C), lambda b, t: (b, t, 0))],
        out_specs=pl.BlockSpec((1, TS // RB, C), lambda b, t: (b, t, 0)),
        out_shape=jax.ShapeDtypeStruct((B, NB1, C), jnp.float32),
        compiler_params=pltpu.CompilerParams(
            dimension_semantics=("parallel", "arbitrary")),
    )(x3)


def _sc_select(g1_all):
    @functools.partial(
        pl.kernel,
        mesh=_MESH,
        out_type=jax.ShapeDtypeStruct((GROUPS, K, L), jnp.int32),
        compiler_params=pltpu.CompilerParams(
            use_tc_tiling_on_sc=False, needs_layout_passes=False),
        scratch_types=[
            pltpu.VMEM((NB1, L), jnp.float32),
            pltpu.VMEM((NB2, L), jnp.float32),
            pltpu.VMEM((K, L), jnp.int32),
        ],
    )
    def k(g1_hbm, bidx_hbm, g1, g2, top_i):
        wid = lax.axis_index("s") * 2 + lax.axis_index("c")
        lane = lax.broadcasted_iota(jnp.int32, (L,), 0)
        neg = jnp.full((L,), -jnp.inf, jnp.float32)
        zero = jnp.zeros((L,), jnp.int32)

        for gi in range(GPW):
            g = wid * GPW + gi
            b = g // CPB
            c0 = (g % CPB) * L

            pltpu.sync_copy(g1_hbm.at[b, :, pl.ds(c0, L)], g1)

            @plsc.parallel_loop(0, NB2, unroll=2)
            def _(ib_):
                base = ib_ * FAN
                m = [jnp.maximum(g1[base + r], g1[base + r + 1])
                     for r in range(0, FAN, 2)]
                while len(m) > 1:
                    m = [jnp.maximum(m[i], m[i + 1])
                         for i in range(0, len(m), 2)]
                g2[ib_] = m[0]

            def sel2(i, carry):
                vs, ids = carry
                return _insert8_idx(vs, ids, g2[i],
                                    jnp.full((L,), i, jnp.int32))

            vs, ids = lax.fori_loop(0, NB2, sel2, ((neg,) * K, (zero,) * K))

            pids = ids
            vs, ids = (neg,) * K, (zero,) * K
            for j in range(K):
                base = pids[j] * FAN

                def child(r, carry):
                    cvs, cids = carry
                    row = base + r
                    v = plsc.load_gather(g1, [row, lane])
                    return _insert8_idx(cvs, cids, v, row)

                vs, ids = lax.fori_loop(0, FAN, child, (vs, ids))

            for j in range(K):
                top_i[j] = ids[j]
            pltpu.sync_copy(top_i, bidx_hbm.at[g])

    return k(g1_all)


def _sc_refetch_fold(x2, bidx):
    @functools.partial(
        pl.kernel,
        mesh=_MESH,
        out_type=jax.ShapeDtypeStruct((GROUPS, K, L), jnp.float32),
        compiler_params=pltpu.CompilerParams(
            use_tc_tiling_on_sc=True, needs_layout_passes=False),
        scratch_types=[
            pltpu.VMEM((K, L), jnp.int32),
            pltpu.VMEM((RPR * L * RB, C), jnp.float32),
            pltpu.VMEM((K, L), jnp.float32),
            pltpu.SemaphoreType.DMA,
            pltpu.SemaphoreType.DMA,
        ],
    )
    def k(x_hbm, bidx_hbm, out_hbm, bidx_v, cand, top_v, semi, semg):
        wid = lax.axis_index("s") * 2 + lax.axis_index("c")
        lane = lax.broadcasted_iota(jnp.int32, (L,), 0)
        neg = jnp.full((L,), -jnp.inf, jnp.float32)

        for gi in range(GPW):
            g = wid * GPW + gi
            b = g // CPB
            c0 = (g % CPB) * L
            row0 = b * S

            pltpu.async_copy(bidx_hbm.at[g], bidx_v, semi).wait()

            rs_a, rs_b = (neg,) * K, (neg,) * K
            for rr in range(K // RPR):
                copies = []
                for jj in range(RPR):
                    idrow = bidx_v[rr * RPR + jj]
                    for l in range(L):
                        blk = idrow[l]
                        copies.append(pltpu.async_copy(
                            x_hbm.at[pl.ds(row0 + blk * RB, RB)],
                            cand.at[pl.ds((jj * L + l) * RB, RB)], semg))
                for cp in copies:
                    cp.wait()

                def fold(t, carry):
                    ra, rb_ = carry
                    jj = t >> 3
                    r2 = (t & 7) * 2
                    base_ = jj * (L * RB) + lane * RB
                    va = plsc.load_gather(cand, [base_ + r2, c0 + lane])
                    vb = plsc.load_gather(cand, [base_ + r2 + 1,
                                                 c0 + lane])
                    return _insert8(ra, va), _insert8(rb_, vb)

                rs_a, rs_b = lax.fori_loop(
                    0, RPR * (RB // 2), fold, (rs_a, rs_b))

            rs = rs_a
            for j in range(K):
                rs = _insert8(rs, rs_b[j])

            for j in range(K):
                top_v[j] = rs[j]
            pltpu.sync_copy(top_v, out_hbm.at[g])

    return k(x2, bidx)


def kernel(inputs):
    x2 = inputs.reshape(B * S, C)
    g1_all = _tc_blockmax(inputs)
    bidx = _sc_select(g1_all)
    out = _sc_refetch_fold(x2, bidx)
    out = out.reshape(B, CPB, K, L).transpose(0, 1, 3, 2)
    return out.reshape(B, C * K)
